# baseline (speedup 1.0000x reference)
import jax
import jax.numpy as jnp
from jax.experimental import pallas as pl
from jax.experimental.pallas import tpu as pltpu


def _gcn_body(nb, bn, x_ref, adj_ref, w_ref, b_ref, out_ref,
              adj_s, cs_s, hht_s, dinv_s):
    p = pl.program_id(0)
    j = pl.program_id(1)

    @pl.when(p == 0)
    def _stage():
        blk = adj_ref[:]
        adj_s[j] = blk.astype(jnp.bfloat16)
        cs_s[0:1, pl.ds(j * bn, bn)] = jnp.sum(blk, axis=0, keepdims=True)

    @pl.when((p == 1) & (j == 0))
    def _prep():
        dinv = jax.lax.rsqrt(2.0 * cs_s[:] + 1.0)
        dinv_s[:] = dinv
        ht = jax.lax.dot_general(w_ref[:], x_ref[:], (((0,), (1,)), ((), ())),
                                 preferred_element_type=jnp.float32)
        hht_s[:] = ht * dinv

    @pl.when(p == 1)
    def _matmul():
        ablk = adj_s[j]
        hht = hht_s[:]
        hi = hht.astype(jnp.bfloat16)
        lo = (hht - hi.astype(jnp.float32)).astype(jnp.bfloat16)
        st = (jnp.dot(hi, ablk, preferred_element_type=jnp.float32) +
              jnp.dot(lo, ablk, preferred_element_type=jnp.float32))
        dj = dinv_s[0:1, pl.ds(j * bn, bn)]
        hhtj = hht_s[:, pl.ds(j * bn, bn)]
        out_ref[:] = jnp.tanh(dj * (2.0 * st + hhtj) + b_ref[:])


def kernel(x, adj, W, b):
    n, in_c = x.shape
    out_c = W.shape[1]
    bn = 256
    nb = n // bn

    import functools
    body = functools.partial(_gcn_body, nb, bn)
    return pl.pallas_call(
        body,
        grid=(2, nb),
        in_specs=[
            pl.BlockSpec((n, in_c), lambda p, j: (0, 0)),
            pl.BlockSpec((n, bn), lambda p, j: (0, j * (1 - p) + (nb - 1) * p)),
            pl.BlockSpec((in_c, out_c), lambda p, j: (0, 0)),
            pl.BlockSpec((out_c, 1), lambda p, j: (0, 0)),
        ],
        out_specs=pl.BlockSpec((out_c, bn), lambda p, j: (0, j)),
        out_shape=jax.ShapeDtypeStruct((out_c, n), jnp.float32),
        scratch_shapes=[
            pltpu.VMEM((nb, n, bn), jnp.bfloat16),
            pltpu.VMEM((1, n), jnp.float32),
            pltpu.VMEM((out_c, n), jnp.float32),
            pltpu.VMEM((1, n), jnp.float32),
        ],
    )(x, adj, W, b.reshape(out_c, 1))


# speedup vs baseline: 1.3864x; 1.3864x over previous
import functools

import jax
import jax.numpy as jnp
from jax.experimental import pallas as pl
from jax.experimental.pallas import tpu as pltpu


def _gcn_body(nstrip, x_ref, adj_hbm, w_ref, b_ref, out_ref, adj_s, sems):
    n = adj_s.shape[0]
    rows = n // nstrip
    copies = [
        pltpu.make_async_copy(
            adj_hbm.at[pl.ds(i * rows, rows), :],
            adj_s.at[pl.ds(i * rows, rows), :],
            sems.at[i],
        )
        for i in range(nstrip)
    ]
    for c in copies:
        c.start()
    ht = jax.lax.dot_general(w_ref[:], x_ref[:], (((0,), (1,)), ((), ())),
                             preferred_element_type=jnp.float32)
    colsum = jnp.zeros((1, n), dtype=jnp.float32)
    for i in range(nstrip):
        copies[i].wait()
        colsum = colsum + jnp.sum(adj_s[i * rows:(i + 1) * rows, :],
                                  axis=0, keepdims=True)
    dinv = jax.lax.rsqrt(2.0 * colsum + 1.0)
    hht = ht * dinv
    st = jax.lax.dot_general(hht, adj_s[:], (((1,), (0,)), ((), ())),
                             preferred_element_type=jnp.float32)
    out_ref[:] = jnp.tanh(dinv * (2.0 * st + hht) + b_ref[:])


def kernel(x, adj, W, b):
    n, in_c = x.shape
    out_c = W.shape[1]
    nstrip = 8
    body = functools.partial(_gcn_body, nstrip)
    return pl.pallas_call(
        body,
        in_specs=[
            pl.BlockSpec((n, in_c), lambda: (0, 0)),
            pl.BlockSpec(memory_space=pltpu.MemorySpace.HBM),
            pl.BlockSpec((in_c, out_c), lambda: (0, 0)),
            pl.BlockSpec((out_c, 1), lambda: (0, 0)),
        ],
        out_specs=pl.BlockSpec((out_c, n), lambda: (0, 0)),
        out_shape=jax.ShapeDtypeStruct((out_c, n), jnp.float32),
        scratch_shapes=[
            pltpu.VMEM((n, n), jnp.float32),
            pltpu.SemaphoreType.DMA((nstrip,)),
        ],
    )(x, adj, W, b.reshape(out_c, 1))
